# Initial kernel scaffold; baseline (speedup 1.0000x reference)
import jax
import jax.numpy as jnp
from jax.experimental import pallas as pl


def kernel(x_num, candidate_x, candidate_y, W0, b0, Wa1, ba1, Wb1, bb1, Wa2, ba2, Wb2, bb2, Wlbl, blbl, Wq, Wk, Wv, Wo, Wh, bh):
    raise NotImplementedError("write your pallas kernel here")


# speedup vs baseline: 1.0114x; 1.0114x over previous
import functools

import jax
import jax.numpy as jnp
import numpy as np
from jax.experimental import pallas as pl
from jax.experimental.pallas import tpu as pltpu

B = 1024
N = 65536
D_IN = 256
D = 512
D_HID = 1024
H = 8
DH = D // H
C = 96

CHUNK = 128
NCHUNK = N // CHUNK
CB = 2048
GB = 64
NEG = float("-inf")


def _mlp(x, W0, b0, Wa1, ba1, Wb1, bb1, Wa2, ba2, Wb2, bb2):
    h = jnp.dot(x, W0, preferred_element_type=jnp.float32) + b0
    z = jnp.maximum(jnp.dot(h, Wa1, preferred_element_type=jnp.float32) + ba1, 0.0)
    h = h + jnp.dot(z, Wb1, preferred_element_type=jnp.float32) + bb1
    z = jnp.maximum(jnp.dot(h, Wa2, preferred_element_type=jnp.float32) + ba2, 0.0)
    h = h + jnp.dot(z, Wb2, preferred_element_type=jnp.float32) + bb2
    return h


def _a_body(x, W0, b0, Wa1, ba1, Wb1, bb1, Wa2, ba2, Wb2, bb2,
            Wq, Wk, Wv, Wlbl, blbl,
            q_o, qh_o, wlk_o, wlv_o, bk_o, bv_o):
    q = _mlp(x[...], W0[...], b0[...], Wa1[...], ba1[...], Wb1[...],
             bb1[...], Wa2[...], ba2[...], Wb2[...], bb2[...])
    q_o[...] = q
    qh_o[...] = jnp.dot(q, Wq[...], preferred_element_type=jnp.float32)
    wlk_o[...] = jnp.dot(Wlbl[...], Wk[...], preferred_element_type=jnp.float32)
    wlv_o[...] = jnp.dot(Wlbl[...], Wv[...], preferred_element_type=jnp.float32)
    bk_o[...] = jnp.dot(blbl[...], Wk[...], preferred_element_type=jnp.float32)
    bv_o[...] = jnp.dot(blbl[...], Wv[...], preferred_element_type=jnp.float32)


def _run_a(x_num, W0, b0, Wa1, ba1, Wb1, bb1, Wa2, ba2, Wb2, bb2,
           Wq, Wk, Wv, Wlbl, blbl):
    outs = [
        jax.ShapeDtypeStruct((B, D), jnp.float32),
        jax.ShapeDtypeStruct((B, D), jnp.float32),
        jax.ShapeDtypeStruct((1, D), jnp.float32),
        jax.ShapeDtypeStruct((1, D), jnp.float32),
        jax.ShapeDtypeStruct((1, D), jnp.float32),
        jax.ShapeDtypeStruct((1, D), jnp.float32),
    ]
    return pl.pallas_call(_a_body, out_shape=outs)(
        x_num, W0, b0, Wa1, ba1, Wb1, bb1, Wa2, ba2, Wb2, bb2,
        Wq, Wk, Wv, Wlbl, blbl)


def _b_body(xc, q, W0, b0, Wa1, ba1, Wb1, bb1, Wa2, ba2, Wb2, bb2,
            kc_o, sims_o, cmax_o):
    kc = _mlp(xc[...], W0[...], b0[...], Wa1[...], ba1[...], Wb1[...],
              bb1[...], Wa2[...], ba2[...], Wb2[...], bb2[...])
    kc_o[...] = kc
    sims = jax.lax.dot_general(q[...], kc, (((1,), (1,)), ((), ())),
                               preferred_element_type=jnp.float32)
    sims_o[...] = sims
    cmax_o[...] = jnp.max(sims.reshape(B, CB // CHUNK, CHUNK), axis=-1)[None]


def _run_b(candidate_x, q, W0, b0, Wa1, ba1, Wb1, bb1, Wa2, ba2, Wb2, bb2):
    nsteps = N // CB
    full = lambda r, c: pl.BlockSpec((r, c), lambda i: (0, 0))
    in_specs = [
        pl.BlockSpec((CB, D_IN), lambda i: (i, 0)),
        full(B, D),
        full(D_IN, D), full(1, D),
        full(D, D_HID), full(1, D_HID), full(D_HID, D), full(1, D),
        full(D, D_HID), full(1, D_HID), full(D_HID, D), full(1, D),
    ]
    out_specs = [
        pl.BlockSpec((CB, D), lambda i: (i, 0)),
        pl.BlockSpec((B, CB), lambda i: (0, i)),
        pl.BlockSpec((1, B, CB // CHUNK), lambda i: (i, 0, 0)),
    ]
    outs = [
        jax.ShapeDtypeStruct((N, D), jnp.float32),
        jax.ShapeDtypeStruct((B, N), jnp.float32),
        jax.ShapeDtypeStruct((nsteps, B, CB // CHUNK), jnp.float32),
    ]
    return pl.pallas_call(
        _b_body, grid=(nsteps,), in_specs=in_specs, out_specs=out_specs,
        out_shape=outs,
        compiler_params=pltpu.CompilerParams(
            dimension_semantics=("arbitrary",)),
    )(candidate_x, q, W0, b0, Wa1, ba1, Wb1, bb1, Wa2, ba2, Wb2, bb2)


def _c_body(cmax, gid_o, t0_o):
    cols = jax.lax.broadcasted_iota(jnp.int32, (B, NCHUNK), 1)
    rows = jax.lax.broadcasted_iota(jnp.int32, (B, 1), 0)

    def step(t, vals):
        m = jnp.max(vals, axis=1, keepdims=True)
        eq = vals == m
        pos = jnp.min(jnp.where(eq, cols, NCHUNK), axis=1, keepdims=True)
        gid_o[:, pl.ds(t, 1)] = pos + rows * NCHUNK
        t0_o[...] = m
        return jnp.where(cols == pos, NEG, vals)

    jax.lax.fori_loop(0, C, step, cmax[...])


def _run_c(cmax):
    outs = [
        jax.ShapeDtypeStruct((B, C), jnp.int32),
        jax.ShapeDtypeStruct((B, 1), jnp.float32),
    ]
    return pl.pallas_call(_c_body, out_shape=outs)(cmax)


def _e_body(sv, sn, idx_o):
    SS = sv.shape[1]
    cols = jax.lax.broadcasted_iota(jnp.int32, (B, SS), 1)

    def step(t, vals):
        m = jnp.max(vals, axis=1, keepdims=True)
        eq = vals == m
        pos = jnp.min(jnp.where(eq, cols, SS), axis=1, keepdims=True)
        sel = jnp.min(jnp.where(cols == pos, sn[...], N), axis=1, keepdims=True)
        idx_o[:, pl.ds(t, 1)] = sel
        return jnp.where(cols == pos, NEG, vals)

    jax.lax.fori_loop(0, C, step, sv[...])


def _run_e(sv, sn):
    return pl.pallas_call(
        _e_body, out_shape=jax.ShapeDtypeStruct((B, C), jnp.int32))(sv, sn)


def _g_body(ctx3, y2, qh, q, wlk, wlv, bk, bv, Wk, Wv, Wo, Wh, bh, out_o):
    ctx = ctx3[...].reshape(C * GB, D)
    kh0 = jnp.dot(ctx, Wk[...], preferred_element_type=jnp.float32) + bk[...]
    vh0 = jnp.dot(ctx, Wv[...], preferred_element_type=jnp.float32) + bv[...]

    r0 = jax.lax.broadcasted_iota(jnp.int32, (C * GB, GB), 0)
    r1 = jax.lax.broadcasted_iota(jnp.int32, (C * GB, GB), 1)
    R = (r0 % GB == r1).astype(jnp.float32)
    g0 = jax.lax.broadcasted_iota(jnp.int32, (D, H), 0)
    g1 = jax.lax.broadcasted_iota(jnp.int32, (D, H), 1)
    Gm = (g0 // DH == g1).astype(jnp.float32)
    gt0 = jax.lax.broadcasted_iota(jnp.int32, (H, D), 0)
    gt1 = jax.lax.broadcasted_iota(jnp.int32, (H, D), 1)
    GmT = (gt1 // DH == gt0).astype(jnp.float32)
    rt0 = jax.lax.broadcasted_iota(jnp.int32, (GB, C * GB), 0)
    rt1 = jax.lax.broadcasted_iota(jnp.int32, (GB, C * GB), 1)
    RT = (rt1 % GB == rt0).astype(jnp.float32)

    qhv = qh[...]
    qh_exp = jnp.dot(R, qhv, preferred_element_type=jnp.float32)
    L0 = jnp.dot(kh0 * qh_exp, Gm, preferred_element_type=jnp.float32)
    A_ = jnp.dot(qhv * wlk[...], Gm, preferred_element_type=jnp.float32)
    A_exp = jnp.dot(R, A_, preferred_element_type=jnp.float32)
    y3 = y2[0]
    Ly = (y3[:, :, None] * A_exp.reshape(C, GB, H)).reshape(C * GB, H)
    L = (L0 + Ly) * (1.0 / np.sqrt(DH))

    L3 = L.reshape(C, GB, H)
    M = jnp.max(L3, axis=0)
    P = jnp.exp(L3 - M[None])
    Ssum = jnp.sum(P, axis=0)
    W3 = P / Ssum[None]
    W2 = W3.reshape(C * GB, H)
    w_exp = jnp.dot(W2, GmT, preferred_element_type=jnp.float32)
    wy = (w_exp.reshape(C, GB, D) * y3[:, :, None]).reshape(C * GB, D)
    Z = w_exp * vh0 + wy * wlv[...]
    o = jnp.dot(RT, Z, preferred_element_type=jnp.float32)
    hq = q[...] + jnp.dot(o, Wo[...], preferred_element_type=jnp.float32)
    out_o[...] = jnp.dot(hq, Wh[...], preferred_element_type=jnp.float32) + bh[...]


def _run_g(ctx3, y2, qh, q, wlk, wlv, bk, bv, Wk, Wv, Wo, Wh, bh):
    nsteps = B // GB
    full = lambda r, c: pl.BlockSpec((r, c), lambda j: (0, 0))
    in_specs = [
        pl.BlockSpec((C, GB, D), lambda j: (0, j, 0)),
        pl.BlockSpec((1, C, GB), lambda j: (j, 0, 0)),
        pl.BlockSpec((GB, D), lambda j: (j, 0)),
        pl.BlockSpec((GB, D), lambda j: (j, 0)),
        full(1, D), full(1, D), full(1, D), full(1, D),
        full(D, D), full(D, D), full(D, D), full(D, 1), full(1, 1),
    ]
    return pl.pallas_call(
        _g_body, grid=(nsteps,), in_specs=in_specs,
        out_specs=pl.BlockSpec((GB, 1), lambda j: (j, 0)),
        out_shape=jax.ShapeDtypeStruct((B, 1), jnp.float32),
        compiler_params=pltpu.CompilerParams(
            dimension_semantics=("arbitrary",)),
    )(ctx3, y2, qh, q, wlk, wlv, bk, bv, Wk, Wv, Wo, Wh, bh)


def kernel(x_num, candidate_x, candidate_y, W0, b0, Wa1, ba1, Wb1, bb1,
           Wa2, ba2, Wb2, bb2, Wlbl, blbl, Wq, Wk, Wv, Wo, Wh, bh):
    r2 = lambda v: v.reshape(1, -1)
    b0r, ba1r, bb1r, ba2r, bb2r = r2(b0), r2(ba1), r2(bb1), r2(ba2), r2(bb2)

    q, qh, wlk, wlv, bk, bv = _run_a(
        x_num, W0, b0r, Wa1, ba1r, Wb1, bb1r, Wa2, ba2r, Wb2, bb2r,
        Wq, Wk, Wv, Wlbl, r2(blbl))

    kc, sims, cmax3 = _run_b(
        candidate_x, q, W0, b0r, Wa1, ba1r, Wb1, bb1r, Wa2, ba2r, Wb2, bb2r)
    cmax = cmax3.transpose(1, 0, 2).reshape(B, NCHUNK)

    _, idx = jax.lax.top_k(sims, C)
    ctx3 = jnp.take(kc, idx, axis=0).transpose(1, 0, 2)
    y2 = jnp.take(candidate_y, idx, axis=0).T
    y2 = y2.reshape(C, B // GB, GB).transpose(1, 0, 2)

    return _run_g(ctx3, y2, qh, q, wlk, wlv, bk, bv, Wk, Wv, Wo, Wh,
                  r2(bh))


# speedup vs baseline: 6.6299x; 6.5551x over previous
import functools

import jax
import jax.numpy as jnp
import numpy as np
from jax import lax
from jax.experimental import pallas as pl
from jax.experimental.pallas import tpu as pltpu
from jax.experimental.pallas import tpu_sc as plsc

B = 1024
N = 65536
D_IN = 256
D = 512
D_HID = 1024
H = 8
DH = D // H
C = 96

CHUNK = 128
NCHUNK = N // CHUNK
CB = 2048
GB = 64
NEG = float("-inf")


def _mlp(x, W0, b0, Wa1, ba1, Wb1, bb1, Wa2, ba2, Wb2, bb2):
    h = jnp.dot(x, W0, preferred_element_type=jnp.float32) + b0
    z = jnp.maximum(jnp.dot(h, Wa1, preferred_element_type=jnp.float32) + ba1, 0.0)
    h = h + jnp.dot(z, Wb1, preferred_element_type=jnp.float32) + bb1
    z = jnp.maximum(jnp.dot(h, Wa2, preferred_element_type=jnp.float32) + ba2, 0.0)
    h = h + jnp.dot(z, Wb2, preferred_element_type=jnp.float32) + bb2
    return h


def _a_body(x, W0, b0, Wa1, ba1, Wb1, bb1, Wa2, ba2, Wb2, bb2,
            Wq, Wk, Wv, Wlbl, blbl,
            q_o, qh_o, wlk_o, wlv_o, bk_o, bv_o):
    q = _mlp(x[...], W0[...], b0[...], Wa1[...], ba1[...], Wb1[...],
             bb1[...], Wa2[...], ba2[...], Wb2[...], bb2[...])
    q_o[...] = q
    qh_o[...] = jnp.dot(q, Wq[...], preferred_element_type=jnp.float32)
    wlk_o[...] = jnp.dot(Wlbl[...], Wk[...], preferred_element_type=jnp.float32)
    wlv_o[...] = jnp.dot(Wlbl[...], Wv[...], preferred_element_type=jnp.float32)
    bk_o[...] = jnp.dot(blbl[...], Wk[...], preferred_element_type=jnp.float32)
    bv_o[...] = jnp.dot(blbl[...], Wv[...], preferred_element_type=jnp.float32)


def _run_a(x_num, W0, b0, Wa1, ba1, Wb1, bb1, Wa2, ba2, Wb2, bb2,
           Wq, Wk, Wv, Wlbl, blbl):
    outs = [
        jax.ShapeDtypeStruct((B, D), jnp.float32),
        jax.ShapeDtypeStruct((B, D), jnp.float32),
        jax.ShapeDtypeStruct((1, D), jnp.float32),
        jax.ShapeDtypeStruct((1, D), jnp.float32),
        jax.ShapeDtypeStruct((1, D), jnp.float32),
        jax.ShapeDtypeStruct((1, D), jnp.float32),
    ]
    return pl.pallas_call(_a_body, out_shape=outs)(
        x_num, W0, b0, Wa1, ba1, Wb1, bb1, Wa2, ba2, Wb2, bb2,
        Wq, Wk, Wv, Wlbl, blbl)


def _b_body(xc, q, W0, b0, Wa1, ba1, Wb1, bb1, Wa2, ba2, Wb2, bb2,
            kc_o, sims_o, cmax_o):
    kc = _mlp(xc[...], W0[...], b0[...], Wa1[...], ba1[...], Wb1[...],
              bb1[...], Wa2[...], ba2[...], Wb2[...], bb2[...])
    kc_o[...] = kc
    sims = jax.lax.dot_general(q[...], kc, (((1,), (1,)), ((), ())),
                               preferred_element_type=jnp.float32)
    sims_o[...] = sims
    cmax_o[...] = jnp.max(sims.reshape(B, CB // CHUNK, CHUNK), axis=-1)[None]


def _run_b(candidate_x, q, W0, b0, Wa1, ba1, Wb1, bb1, Wa2, ba2, Wb2, bb2):
    nsteps = N // CB
    full = lambda r, c: pl.BlockSpec((r, c), lambda i: (0, 0))
    in_specs = [
        pl.BlockSpec((CB, D_IN), lambda i: (i, 0)),
        full(B, D),
        full(D_IN, D), full(1, D),
        full(D, D_HID), full(1, D_HID), full(D_HID, D), full(1, D),
        full(D, D_HID), full(1, D_HID), full(D_HID, D), full(1, D),
    ]
    out_specs = [
        pl.BlockSpec((CB, D), lambda i: (i, 0)),
        pl.BlockSpec((B, CB), lambda i: (0, i)),
        pl.BlockSpec((1, B, CB // CHUNK), lambda i: (i, 0, 0)),
    ]
    outs = [
        jax.ShapeDtypeStruct((N, D), jnp.float32),
        jax.ShapeDtypeStruct((B, N), jnp.float32),
        jax.ShapeDtypeStruct((nsteps, B, CB // CHUNK), jnp.float32),
    ]
    return pl.pallas_call(
        _b_body, grid=(nsteps,), in_specs=in_specs, out_specs=out_specs,
        out_shape=outs,
        compiler_params=pltpu.CompilerParams(
            dimension_semantics=("arbitrary",)),
    )(candidate_x, q, W0, b0, Wa1, ba1, Wb1, bb1, Wa2, ba2, Wb2, bb2)


def _topc_body(W, vals_ref, gid_o):
    cols = jax.lax.broadcasted_iota(jnp.int32, (B, W), 1)
    rows = jax.lax.broadcasted_iota(jnp.int32, (B, 1), 0)
    colsC = jax.lax.broadcasted_iota(jnp.int32, (B, C), 1)

    def step(t, carry):
        vals, acc = carry
        m = jnp.max(vals, axis=1, keepdims=True)
        eq = vals == m
        pos = jnp.min(jnp.where(eq, cols, W), axis=1, keepdims=True)
        acc = jnp.where(colsC == t, pos + rows * W, acc)
        return (jnp.where(cols == pos, NEG, vals), acc)

    acc0 = jnp.zeros((B, C), jnp.int32)
    _, acc = jax.lax.fori_loop(0, C, step, (vals_ref[...], acc0))
    gid_o[...] = acc


def _run_topc(vals, W):
    return pl.pallas_call(
        functools.partial(_topc_body, W),
        out_shape=jax.ShapeDtypeStruct((B, C), jnp.int32))(vals)


S = 256
NW = 32
RW = B // NW


def _dl_body(sims_v, gids_f, gv_o, gid_v, chunk_v, sem):
    wid = lax.axis_index("s") * 2 + lax.axis_index("c")
    r0 = wid * RW

    def row_step(i, _):
        b = r0 + i
        pltpu.sync_copy(gids_f.at[pl.ds(b * C, C)], gid_v)
        pltpu.async_copy(sims_v.at[gid_v], chunk_v, sem).wait()
        pltpu.sync_copy(chunk_v, gv_o.at[pl.ds(b * C, C)])
        return 0

    lax.fori_loop(0, RW, row_step, 0)


def _run_dl(sims_v, gids_f):
    mesh = plsc.VectorSubcoreMesh(core_axis_name="c", subcore_axis_name="s")
    f = pl.kernel(
        _dl_body, mesh=mesh,
        out_type=jax.ShapeDtypeStruct((B * C, CHUNK), jnp.float32),
        scratch_types=[
            pltpu.VMEM((C,), jnp.int32),
            pltpu.VMEM((C, CHUNK), jnp.float32),
            pltpu.SemaphoreType.DMA,
        ])
    return f(sims_v, gids_f)


NSUB = C * CHUNK // 16
EQ = 64


def _e1_body(gv, gid3, smax_o, n3_o):
    j = pl.program_id(0)
    g = gv[...].reshape(EQ, NSUB, 16)
    smax_o[...] = jnp.max(g, axis=-1)
    rowg = jax.lax.broadcasted_iota(jnp.int32, (EQ, C), 0) + j * EQ
    cid2 = gid3[0] - rowg * NCHUNK
    n3_o[...] = (cid2[:, :, None] * CHUNK
                 + jax.lax.broadcasted_iota(jnp.int32, (EQ, C, CHUNK), 2))


def _run_e1(gv2, gid3):
    nsteps = B // EQ
    in_specs = [
        pl.BlockSpec((EQ, C * CHUNK), lambda j: (j, 0)),
        pl.BlockSpec((1, EQ, C), lambda j: (j, 0, 0)),
    ]
    out_specs = [
        pl.BlockSpec((EQ, NSUB), lambda j: (j, 0)),
        pl.BlockSpec((EQ, C, CHUNK), lambda j: (j, 0, 0)),
    ]
    outs = [
        jax.ShapeDtypeStruct((B, NSUB), jnp.float32),
        jax.ShapeDtypeStruct((B, C, CHUNK), jnp.int32),
    ]
    return pl.pallas_call(
        _e1_body, grid=(nsteps,), in_specs=in_specs, out_specs=out_specs,
        out_shape=outs,
        compiler_params=pltpu.CompilerParams(
            dimension_semantics=("arbitrary",)),
    )(gv2, gid3)


KJ2 = 128
BPW2 = (B * C * 16) // NW


def _d2_body(vt, nt, idx16, sv_o, sn_o, idx_v, v_v, n_v, sem, sem2):
    wid = lax.axis_index("s") * 2 + lax.axis_index("c")

    def step(t, _):
        base = wid * BPW2 + t * KJ2
        pltpu.sync_copy(idx16.at[pl.ds(base, KJ2)], idx_v)
        cp1 = pltpu.async_copy(vt.at[idx_v], v_v, sem)
        cp2 = pltpu.async_copy(nt.at[idx_v], n_v, sem2)
        cp1.wait()
        cp2.wait()
        pltpu.sync_copy(v_v, sv_o.at[pl.ds(base, KJ2)])
        pltpu.sync_copy(n_v, sn_o.at[pl.ds(base, KJ2)])
        return 0

    lax.fori_loop(0, BPW2 // KJ2, step, 0)


def _run_d2(vt, nt, idx16):
    mesh = plsc.VectorSubcoreMesh(core_axis_name="c", subcore_axis_name="s")
    f = pl.kernel(
        _d2_body, mesh=mesh,
        out_type=[
            jax.ShapeDtypeStruct((B * C * 16,), jnp.float32),
            jax.ShapeDtypeStruct((B * C * 16,), jnp.int32),
        ],
        scratch_types=[
            pltpu.VMEM((KJ2,), jnp.int32),
            pltpu.VMEM((KJ2,), jnp.float32),
            pltpu.VMEM((KJ2,), jnp.int32),
            pltpu.SemaphoreType.DMA,
            pltpu.SemaphoreType.DMA,
        ])
    return f(vt, nt, idx16)


SS = C * 16


def _e3_body(sv, sn, idx_o):
    cols = jax.lax.broadcasted_iota(jnp.int32, (B, SS), 1)
    colsC = jax.lax.broadcasted_iota(jnp.int32, (B, C), 1)
    snv = sn[...]

    def step(t, carry):
        vals, acc = carry
        m = jnp.max(vals, axis=1, keepdims=True)
        eq = vals == m
        pos = jnp.min(jnp.where(eq, cols, SS), axis=1, keepdims=True)
        sel = jnp.min(jnp.where(cols == pos, snv, N), axis=1, keepdims=True)
        acc = jnp.where(colsC == t, sel, acc)
        return (jnp.where(cols == pos, NEG, vals), acc)

    acc0 = jnp.zeros((B, C), jnp.int32)
    _, acc = jax.lax.fori_loop(0, C, step, (sv[...], acc0))
    idx_o[...] = acc


def _run_e3(sv, sn):
    return pl.pallas_call(
        _e3_body,
        out_shape=jax.ShapeDtypeStruct((B, C), jnp.int32))(sv, sn)


KJ = 128
BPW = (B * C) // NW


def _f_body(kc, y, idxf, ctx_o, yc_o, idx_v, rows_v, y_v, sem, sem2):
    wid = lax.axis_index("s") * 2 + lax.axis_index("c")

    def step(t, _):
        base = wid * BPW + t * KJ
        pltpu.sync_copy(idxf.at[pl.ds(base, KJ)], idx_v)
        cp1 = pltpu.async_copy(kc.at[idx_v], rows_v, sem)
        cp2 = pltpu.async_copy(y.at[idx_v], y_v, sem2)
        cp1.wait()
        cp2.wait()
        pltpu.sync_copy(rows_v, ctx_o.at[pl.ds(base, KJ)])
        pltpu.sync_copy(y_v, yc_o.at[pl.ds(base, KJ)])
        return 0

    lax.fori_loop(0, BPW // KJ, step, 0)


def _run_f(kc, y, idxf):
    mesh = plsc.VectorSubcoreMesh(core_axis_name="c", subcore_axis_name="s")
    f = pl.kernel(
        _f_body, mesh=mesh,
        out_type=[
            jax.ShapeDtypeStruct((B * C, D), jnp.float32),
            jax.ShapeDtypeStruct((B * C,), jnp.float32),
        ],
        scratch_types=[
            pltpu.VMEM((KJ,), jnp.int32),
            pltpu.VMEM((KJ, D), jnp.float32),
            pltpu.VMEM((KJ,), jnp.float32),
            pltpu.SemaphoreType.DMA,
            pltpu.SemaphoreType.DMA,
        ])
    return f(kc, y, idxf)


def _g_body(ctx3, y2, qh, q, wlk, wlv, bk, bv, Wk, Wv, Wo, Wh, bh, out_o):
    ctx = ctx3[...].reshape(C * GB, D)
    kh0 = jnp.dot(ctx, Wk[...], preferred_element_type=jnp.float32) + bk[...]
    vh0 = jnp.dot(ctx, Wv[...], preferred_element_type=jnp.float32) + bv[...]

    r0 = jax.lax.broadcasted_iota(jnp.int32, (C * GB, GB), 0)
    r1 = jax.lax.broadcasted_iota(jnp.int32, (C * GB, GB), 1)
    R = (r0 % GB == r1).astype(jnp.float32)
    g0 = jax.lax.broadcasted_iota(jnp.int32, (D, H), 0)
    g1 = jax.lax.broadcasted_iota(jnp.int32, (D, H), 1)
    Gm = (g0 // DH == g1).astype(jnp.float32)
    gt0 = jax.lax.broadcasted_iota(jnp.int32, (H, D), 0)
    gt1 = jax.lax.broadcasted_iota(jnp.int32, (H, D), 1)
    GmT = (gt1 // DH == gt0).astype(jnp.float32)
    rt0 = jax.lax.broadcasted_iota(jnp.int32, (GB, C * GB), 0)
    rt1 = jax.lax.broadcasted_iota(jnp.int32, (GB, C * GB), 1)
    RT = (rt1 % GB == rt0).astype(jnp.float32)

    qhv = qh[...]
    qh_exp = jnp.dot(R, qhv, preferred_element_type=jnp.float32)
    L0 = jnp.dot(kh0 * qh_exp, Gm, preferred_element_type=jnp.float32)
    A_ = jnp.dot(qhv * wlk[...], Gm, preferred_element_type=jnp.float32)
    A_exp = jnp.dot(R, A_, preferred_element_type=jnp.float32)
    y3 = y2[0]
    Ly = (y3[:, :, None] * A_exp.reshape(C, GB, H)).reshape(C * GB, H)
    L = (L0 + Ly) * (1.0 / np.sqrt(DH))

    L3 = L.reshape(C, GB, H)
    M = jnp.max(L3, axis=0)
    P = jnp.exp(L3 - M[None])
    Ssum = jnp.sum(P, axis=0)
    W3 = P / Ssum[None]
    W2 = W3.reshape(C * GB, H)
    w_exp = jnp.dot(W2, GmT, preferred_element_type=jnp.float32)
    wy = (w_exp.reshape(C, GB, D) * y3[:, :, None]).reshape(C * GB, D)
    Z = w_exp * vh0 + wy * wlv[...]
    o = jnp.dot(RT, Z, preferred_element_type=jnp.float32)
    hq = q[...] + jnp.dot(o, Wo[...], preferred_element_type=jnp.float32)
    out_o[...] = jnp.dot(hq, Wh[...], preferred_element_type=jnp.float32) + bh[...]


def _run_g(ctx3, y2, qh, q, wlk, wlv, bk, bv, Wk, Wv, Wo, Wh, bh):
    nsteps = B // GB
    full = lambda r, c: pl.BlockSpec((r, c), lambda j: (0, 0))
    in_specs = [
        pl.BlockSpec((C, GB, D), lambda j: (0, j, 0)),
        pl.BlockSpec((1, C, GB), lambda j: (j, 0, 0)),
        pl.BlockSpec((GB, D), lambda j: (j, 0)),
        pl.BlockSpec((GB, D), lambda j: (j, 0)),
        full(1, D), full(1, D), full(1, D), full(1, D),
        full(D, D), full(D, D), full(D, D), full(D, 1), full(1, 1),
    ]
    return pl.pallas_call(
        _g_body, grid=(nsteps,), in_specs=in_specs,
        out_specs=pl.BlockSpec((GB, 1), lambda j: (j, 0)),
        out_shape=jax.ShapeDtypeStruct((B, 1), jnp.float32),
        compiler_params=pltpu.CompilerParams(
            dimension_semantics=("arbitrary",)),
    )(ctx3, y2, qh, q, wlk, wlv, bk, bv, Wk, Wv, Wo, Wh, bh)


def kernel(x_num, candidate_x, candidate_y, W0, b0, Wa1, ba1, Wb1, bb1,
           Wa2, ba2, Wb2, bb2, Wlbl, blbl, Wq, Wk, Wv, Wo, Wh, bh):
    r2 = lambda v: v.reshape(1, -1)
    b0r, ba1r, bb1r, ba2r, bb2r = r2(b0), r2(ba1), r2(bb1), r2(ba2), r2(bb2)

    q, qh, wlk, wlv, bk, bv = _run_a(
        x_num, W0, b0r, Wa1, ba1r, Wb1, bb1r, Wa2, ba2r, Wb2, bb2r,
        Wq, Wk, Wv, Wlbl, r2(blbl))

    kc, sims, cmax3 = _run_b(
        candidate_x, q, W0, b0r, Wa1, ba1r, Wb1, bb1r, Wa2, ba2r, Wb2, bb2r)
    cmax = cmax3.transpose(1, 0, 2).reshape(B, NCHUNK)

    gids = _run_topc(cmax, NCHUNK)
    gv = _run_dl(sims.reshape(B * NCHUNK, CHUNK), gids.reshape(-1))
    smax, n3 = _run_e1(gv.reshape(B, C * CHUNK),
                       gids.reshape(B // EQ, EQ, C))
    sgids = _run_topc(smax, NSUB)
    idx16 = (sgids[:, :, None] * 16
             + jnp.arange(16, dtype=jnp.int32)).reshape(-1)
    sv, sn = _run_d2(gv.reshape(-1), n3.reshape(-1), idx16)
    idx = _run_e3(sv.reshape(B, SS), sn.reshape(B, SS))
    idxf = idx.T.reshape(-1)
    ctx_flat, yc = _run_f(kc, candidate_y, idxf)
    ctx3 = ctx_flat.reshape(C, B, D)
    y2 = yc.reshape(C, B // GB, GB).transpose(1, 0, 2)

    return _run_g(ctx3, y2, qh, q, wlk, wlv, bk, bv, Wk, Wv, Wo, Wh,
                  r2(bh))


# speedup vs baseline: 7.6329x; 1.1513x over previous
import functools

import jax
import jax.numpy as jnp
import numpy as np
from jax import lax
from jax.experimental import pallas as pl
from jax.experimental.pallas import tpu as pltpu
from jax.experimental.pallas import tpu_sc as plsc

B = 1024
N = 65536
D_IN = 256
D = 512
D_HID = 1024
H = 8
DH = D // H
C = 96

CHUNK = 128
NCHUNK = N // CHUNK
CB = 2048
GB = 64
NEG = float("-inf")


def _mlp(x, W0, b0, Wa1, ba1, Wb1, bb1, Wa2, ba2, Wb2, bb2):
    h = jnp.dot(x, W0, preferred_element_type=jnp.float32) + b0
    z = jnp.maximum(jnp.dot(h, Wa1, preferred_element_type=jnp.float32) + ba1, 0.0)
    h = h + jnp.dot(z, Wb1, preferred_element_type=jnp.float32) + bb1
    z = jnp.maximum(jnp.dot(h, Wa2, preferred_element_type=jnp.float32) + ba2, 0.0)
    h = h + jnp.dot(z, Wb2, preferred_element_type=jnp.float32) + bb2
    return h


def _a_body(x, W0, b0, Wa1, ba1, Wb1, bb1, Wa2, ba2, Wb2, bb2,
            Wq, Wk, Wv, Wlbl, blbl,
            q_o, qh_o, wlk_o, wlv_o, bk_o, bv_o):
    q = _mlp(x[...], W0[...], b0[...], Wa1[...], ba1[...], Wb1[...],
             bb1[...], Wa2[...], ba2[...], Wb2[...], bb2[...])
    q_o[...] = q
    qh_o[...] = jnp.dot(q, Wq[...], preferred_element_type=jnp.float32)
    wlk_o[...] = jnp.dot(Wlbl[...], Wk[...], preferred_element_type=jnp.float32)
    wlv_o[...] = jnp.dot(Wlbl[...], Wv[...], preferred_element_type=jnp.float32)
    bk_o[...] = jnp.dot(blbl[...], Wk[...], preferred_element_type=jnp.float32)
    bv_o[...] = jnp.dot(blbl[...], Wv[...], preferred_element_type=jnp.float32)


def _run_a(x_num, W0, b0, Wa1, ba1, Wb1, bb1, Wa2, ba2, Wb2, bb2,
           Wq, Wk, Wv, Wlbl, blbl):
    outs = [
        jax.ShapeDtypeStruct((B, D), jnp.float32),
        jax.ShapeDtypeStruct((B, D), jnp.float32),
        jax.ShapeDtypeStruct((1, D), jnp.float32),
        jax.ShapeDtypeStruct((1, D), jnp.float32),
        jax.ShapeDtypeStruct((1, D), jnp.float32),
        jax.ShapeDtypeStruct((1, D), jnp.float32),
    ]
    return pl.pallas_call(_a_body, out_shape=outs)(
        x_num, W0, b0, Wa1, ba1, Wb1, bb1, Wa2, ba2, Wb2, bb2,
        Wq, Wk, Wv, Wlbl, blbl)


def _b_body(xc, q, W0, b0, Wa1, ba1, Wb1, bb1, Wa2, ba2, Wb2, bb2,
            kc_o, sims_o, cmax_o):
    kc = _mlp(xc[...], W0[...], b0[...], Wa1[...], ba1[...], Wb1[...],
              bb1[...], Wa2[...], ba2[...], Wb2[...], bb2[...])
    kc_o[...] = kc
    sims = jax.lax.dot_general(q[...], kc, (((1,), (1,)), ((), ())),
                               preferred_element_type=jnp.float32)
    sims_o[...] = sims
    cmax_o[...] = jnp.max(sims.reshape(B, CB // CHUNK, CHUNK), axis=-1)[None]


def _run_b(candidate_x, q, W0, b0, Wa1, ba1, Wb1, bb1, Wa2, ba2, Wb2, bb2):
    nsteps = N // CB
    full = lambda r, c: pl.BlockSpec((r, c), lambda i: (0, 0))
    in_specs = [
        pl.BlockSpec((CB, D_IN), lambda i: (i, 0)),
        full(B, D),
        full(D_IN, D), full(1, D),
        full(D, D_HID), full(1, D_HID), full(D_HID, D), full(1, D),
        full(D, D_HID), full(1, D_HID), full(D_HID, D), full(1, D),
    ]
    out_specs = [
        pl.BlockSpec((CB, D), lambda i: (i, 0)),
        pl.BlockSpec((B, CB), lambda i: (0, i)),
        pl.BlockSpec((1, B, CB // CHUNK), lambda i: (i, 0, 0)),
    ]
    outs = [
        jax.ShapeDtypeStruct((N, D), jnp.float32),
        jax.ShapeDtypeStruct((B, N), jnp.float32),
        jax.ShapeDtypeStruct((nsteps, B, CB // CHUNK), jnp.float32),
    ]
    return pl.pallas_call(
        _b_body, grid=(nsteps,), in_specs=in_specs, out_specs=out_specs,
        out_shape=outs,
        compiler_params=pltpu.CompilerParams(
            dimension_semantics=("arbitrary",)),
    )(candidate_x, q, W0, b0, Wa1, ba1, Wb1, bb1, Wa2, ba2, Wb2, bb2)


def _topc_body(W, vals_ref, gid_o):
    cols = jax.lax.broadcasted_iota(jnp.int32, (B, W), 1)
    rows = jax.lax.broadcasted_iota(jnp.int32, (B, 1), 0)
    colsC = jax.lax.broadcasted_iota(jnp.int32, (B, C), 1)

    def step(t, carry):
        vals, acc = carry
        m = jnp.max(vals, axis=1, keepdims=True)
        eq = vals == m
        pos = jnp.min(jnp.where(eq, cols, W), axis=1, keepdims=True)
        acc = jnp.where(colsC == t, pos + rows * W, acc)
        return (jnp.where(cols == pos, NEG, vals), acc)

    acc0 = jnp.zeros((B, C), jnp.int32)
    _, acc = jax.lax.fori_loop(0, C, step, (vals_ref[...], acc0))
    gid_o[...] = acc


def _run_topc(vals, W):
    return pl.pallas_call(
        functools.partial(_topc_body, W),
        out_shape=jax.ShapeDtypeStruct((B, C), jnp.int32))(vals)


S = 256
NW = 32
RW = B // NW


def _dl_body(sims_v, gids_f, gv_o, gid_v, chunk_v, sem):
    wid = lax.axis_index("s") * 2 + lax.axis_index("c")
    r0 = wid * RW

    def row_step(i, _):
        b = r0 + i
        pltpu.sync_copy(gids_f.at[pl.ds(b * C, C)], gid_v)
        pltpu.async_copy(sims_v.at[gid_v], chunk_v, sem).wait()
        pltpu.sync_copy(chunk_v, gv_o.at[pl.ds(b * C, C)])
        return 0

    lax.fori_loop(0, RW, row_step, 0)


def _run_dl(sims_v, gids_f):
    mesh = plsc.VectorSubcoreMesh(core_axis_name="c", subcore_axis_name="s")
    f = pl.kernel(
        _dl_body, mesh=mesh,
        out_type=jax.ShapeDtypeStruct((B * C, CHUNK), jnp.float32),
        scratch_types=[
            pltpu.VMEM((C,), jnp.int32),
            pltpu.VMEM((C, CHUNK), jnp.float32),
            pltpu.SemaphoreType.DMA,
        ])
    return f(sims_v, gids_f)


NSUB = C * CHUNK // 16
EQ = 64


def _e1_body(gv, gid3, smax_o, n3_o):
    j = pl.program_id(0)
    g = gv[...].reshape(EQ, NSUB, 16)
    smax_o[...] = jnp.max(g, axis=-1)
    rowg = jax.lax.broadcasted_iota(jnp.int32, (EQ, C), 0) + j * EQ
    cid2 = gid3[0] - rowg * NCHUNK
    n3_o[...] = (cid2[:, :, None] * CHUNK
                 + jax.lax.broadcasted_iota(jnp.int32, (EQ, C, CHUNK), 2))


def _run_e1(gv2, gid3):
    nsteps = B // EQ
    in_specs = [
        pl.BlockSpec((EQ, C * CHUNK), lambda j: (j, 0)),
        pl.BlockSpec((1, EQ, C), lambda j: (j, 0, 0)),
    ]
    out_specs = [
        pl.BlockSpec((EQ, NSUB), lambda j: (j, 0)),
        pl.BlockSpec((EQ, C, CHUNK), lambda j: (j, 0, 0)),
    ]
    outs = [
        jax.ShapeDtypeStruct((B, NSUB), jnp.float32),
        jax.ShapeDtypeStruct((B, C, CHUNK), jnp.int32),
    ]
    return pl.pallas_call(
        _e1_body, grid=(nsteps,), in_specs=in_specs, out_specs=out_specs,
        out_shape=outs,
        compiler_params=pltpu.CompilerParams(
            dimension_semantics=("arbitrary",)),
    )(gv2, gid3)


KJ2 = 128
FIRE = 12
KO2 = KJ2 * FIRE
BPW2 = (B * C * 16) // NW


def _d2_body(vt, nt, idx16, sv_o, sn_o, idx_v, v_v, n_v, sem, sem2):
    wid = lax.axis_index("s") * 2 + lax.axis_index("c")

    def step(t, _):
        base = wid * BPW2 + t * KO2
        pltpu.sync_copy(idx16.at[pl.ds(base, KO2)], idx_v)
        cps = []
        for bk in range(FIRE):
            sl = pl.ds(bk * KJ2, KJ2)
            cps.append(pltpu.async_copy(vt.at[idx_v.at[sl]], v_v.at[sl], sem))
            cps.append(pltpu.async_copy(nt.at[idx_v.at[sl]], n_v.at[sl], sem2))
        for cp in cps:
            cp.wait()
        pltpu.sync_copy(v_v, sv_o.at[pl.ds(base, KO2)])
        pltpu.sync_copy(n_v, sn_o.at[pl.ds(base, KO2)])
        return 0

    lax.fori_loop(0, BPW2 // KO2, step, 0)


def _run_d2(vt, nt, idx16):
    mesh = plsc.VectorSubcoreMesh(core_axis_name="c", subcore_axis_name="s")
    f = pl.kernel(
        _d2_body, mesh=mesh,
        out_type=[
            jax.ShapeDtypeStruct((B * C * 16,), jnp.float32),
            jax.ShapeDtypeStruct((B * C * 16,), jnp.int32),
        ],
        scratch_types=[
            pltpu.VMEM((KO2,), jnp.int32),
            pltpu.VMEM((KO2,), jnp.float32),
            pltpu.VMEM((KO2,), jnp.int32),
            pltpu.SemaphoreType.DMA,
            pltpu.SemaphoreType.DMA,
        ])
    return f(vt, nt, idx16)


SS = C * 16


def _e3_body(sv, sn, idx_o):
    cols = jax.lax.broadcasted_iota(jnp.int32, (B, SS), 1)
    colsC = jax.lax.broadcasted_iota(jnp.int32, (B, C), 1)
    snv = sn[...]

    def step(t, carry):
        vals, acc = carry
        m = jnp.max(vals, axis=1, keepdims=True)
        eq = vals == m
        pos = jnp.min(jnp.where(eq, cols, SS), axis=1, keepdims=True)
        sel = jnp.min(jnp.where(cols == pos, snv, N), axis=1, keepdims=True)
        acc = jnp.where(colsC == t, sel, acc)
        return (jnp.where(cols == pos, NEG, vals), acc)

    acc0 = jnp.zeros((B, C), jnp.int32)
    _, acc = jax.lax.fori_loop(0, C, step, (sv[...], acc0))
    idx_o[...] = acc


def _run_e3(sv, sn):
    return pl.pallas_call(
        _e3_body,
        out_shape=jax.ShapeDtypeStruct((B, C), jnp.int32))(sv, sn)


KJ = 128
BPW = (B * C) // NW


def _f_body(kc, y, idxf, ctx_o, yc_o, idx_v, rows_v, y_v, sem, sem2):
    wid = lax.axis_index("s") * 2 + lax.axis_index("c")

    def step(t, _):
        base = wid * BPW + t * KJ
        pltpu.sync_copy(idxf.at[pl.ds(base, KJ)], idx_v)
        cp1 = pltpu.async_copy(kc.at[idx_v], rows_v, sem)
        cp2 = pltpu.async_copy(y.at[idx_v], y_v, sem2)
        cp1.wait()
        cp2.wait()
        pltpu.sync_copy(rows_v, ctx_o.at[pl.ds(base, KJ)])
        pltpu.sync_copy(y_v, yc_o.at[pl.ds(base, KJ)])
        return 0

    lax.fori_loop(0, BPW // KJ, step, 0)


def _run_f(kc, y, idxf):
    mesh = plsc.VectorSubcoreMesh(core_axis_name="c", subcore_axis_name="s")
    f = pl.kernel(
        _f_body, mesh=mesh,
        out_type=[
            jax.ShapeDtypeStruct((B * C, D), jnp.float32),
            jax.ShapeDtypeStruct((B * C,), jnp.float32),
        ],
        scratch_types=[
            pltpu.VMEM((KJ,), jnp.int32),
            pltpu.VMEM((KJ, D), jnp.float32),
            pltpu.VMEM((KJ,), jnp.float32),
            pltpu.SemaphoreType.DMA,
            pltpu.SemaphoreType.DMA,
        ])
    return f(kc, y, idxf)


def _g_body(ctx3, y2, qh, q, wlk, wlv, bk, bv, Wk, Wv, Wo, Wh, bh, out_o):
    ctx = ctx3[...].reshape(C * GB, D)
    kh0 = jnp.dot(ctx, Wk[...], preferred_element_type=jnp.float32) + bk[...]
    vh0 = jnp.dot(ctx, Wv[...], preferred_element_type=jnp.float32) + bv[...]

    r0 = jax.lax.broadcasted_iota(jnp.int32, (C * GB, GB), 0)
    r1 = jax.lax.broadcasted_iota(jnp.int32, (C * GB, GB), 1)
    R = (r0 % GB == r1).astype(jnp.float32)
    g0 = jax.lax.broadcasted_iota(jnp.int32, (D, H), 0)
    g1 = jax.lax.broadcasted_iota(jnp.int32, (D, H), 1)
    Gm = (g0 // DH == g1).astype(jnp.float32)
    gt0 = jax.lax.broadcasted_iota(jnp.int32, (H, D), 0)
    gt1 = jax.lax.broadcasted_iota(jnp.int32, (H, D), 1)
    GmT = (gt1 // DH == gt0).astype(jnp.float32)
    rt0 = jax.lax.broadcasted_iota(jnp.int32, (GB, C * GB), 0)
    rt1 = jax.lax.broadcasted_iota(jnp.int32, (GB, C * GB), 1)
    RT = (rt1 % GB == rt0).astype(jnp.float32)

    qhv = qh[...]
    qh_exp = jnp.dot(R, qhv, preferred_element_type=jnp.float32)
    L0 = jnp.dot(kh0 * qh_exp, Gm, preferred_element_type=jnp.float32)
    A_ = jnp.dot(qhv * wlk[...], Gm, preferred_element_type=jnp.float32)
    A_exp = jnp.dot(R, A_, preferred_element_type=jnp.float32)
    y3 = y2[0]
    Ly = (y3[:, :, None] * A_exp.reshape(C, GB, H)).reshape(C * GB, H)
    L = (L0 + Ly) * (1.0 / np.sqrt(DH))

    L3 = L.reshape(C, GB, H)
    M = jnp.max(L3, axis=0)
    P = jnp.exp(L3 - M[None])
    Ssum = jnp.sum(P, axis=0)
    W3 = P / Ssum[None]
    W2 = W3.reshape(C * GB, H)
    w_exp = jnp.dot(W2, GmT, preferred_element_type=jnp.float32)
    wy = (w_exp.reshape(C, GB, D) * y3[:, :, None]).reshape(C * GB, D)
    Z = w_exp * vh0 + wy * wlv[...]
    o = jnp.dot(RT, Z, preferred_element_type=jnp.float32)
    hq = q[...] + jnp.dot(o, Wo[...], preferred_element_type=jnp.float32)
    out_o[...] = jnp.dot(hq, Wh[...], preferred_element_type=jnp.float32) + bh[...]


def _run_g(ctx3, y2, qh, q, wlk, wlv, bk, bv, Wk, Wv, Wo, Wh, bh):
    nsteps = B // GB
    full = lambda r, c: pl.BlockSpec((r, c), lambda j: (0, 0))
    in_specs = [
        pl.BlockSpec((C, GB, D), lambda j: (0, j, 0)),
        pl.BlockSpec((1, C, GB), lambda j: (j, 0, 0)),
        pl.BlockSpec((GB, D), lambda j: (j, 0)),
        pl.BlockSpec((GB, D), lambda j: (j, 0)),
        full(1, D), full(1, D), full(1, D), full(1, D),
        full(D, D), full(D, D), full(D, D), full(D, 1), full(1, 1),
    ]
    return pl.pallas_call(
        _g_body, grid=(nsteps,), in_specs=in_specs,
        out_specs=pl.BlockSpec((GB, 1), lambda j: (j, 0)),
        out_shape=jax.ShapeDtypeStruct((B, 1), jnp.float32),
        compiler_params=pltpu.CompilerParams(
            dimension_semantics=("arbitrary",)),
    )(ctx3, y2, qh, q, wlk, wlv, bk, bv, Wk, Wv, Wo, Wh, bh)


def kernel(x_num, candidate_x, candidate_y, W0, b0, Wa1, ba1, Wb1, bb1,
           Wa2, ba2, Wb2, bb2, Wlbl, blbl, Wq, Wk, Wv, Wo, Wh, bh):
    r2 = lambda v: v.reshape(1, -1)
    b0r, ba1r, bb1r, ba2r, bb2r = r2(b0), r2(ba1), r2(bb1), r2(ba2), r2(bb2)

    q, qh, wlk, wlv, bk, bv = _run_a(
        x_num, W0, b0r, Wa1, ba1r, Wb1, bb1r, Wa2, ba2r, Wb2, bb2r,
        Wq, Wk, Wv, Wlbl, r2(blbl))

    kc, sims, cmax3 = _run_b(
        candidate_x, q, W0, b0r, Wa1, ba1r, Wb1, bb1r, Wa2, ba2r, Wb2, bb2r)
    cmax = cmax3.transpose(1, 0, 2).reshape(B, NCHUNK)

    gids = _run_topc(cmax, NCHUNK)
    gv = _run_dl(sims.reshape(B * NCHUNK, CHUNK), gids.reshape(-1))
    smax, n3 = _run_e1(gv.reshape(B, C * CHUNK),
                       gids.reshape(B // EQ, EQ, C))
    sgids = _run_topc(smax, NSUB)
    idx16 = (sgids[:, :, None] * 16
             + jnp.arange(16, dtype=jnp.int32)).reshape(-1)
    sv, sn = _run_d2(gv.reshape(-1), n3.reshape(-1), idx16)
    idx = _run_e3(sv.reshape(B, SS), sn.reshape(B, SS))
    idxf = idx.T.reshape(-1)
    ctx_flat, yc = _run_f(kc, candidate_y, idxf)
    ctx3 = ctx_flat.reshape(C, B, D)
    y2 = yc.reshape(C, B // GB, GB).transpose(1, 0, 2)

    return _run_g(ctx3, y2, qh, q, wlk, wlv, bk, bv, Wk, Wv, Wo, Wh,
                  r2(bh))


# speedup vs baseline: 7.6689x; 1.0047x over previous
import functools

import jax
import jax.numpy as jnp
import numpy as np
from jax import lax
from jax.experimental import pallas as pl
from jax.experimental.pallas import tpu as pltpu
from jax.experimental.pallas import tpu_sc as plsc

B = 1024
N = 65536
D_IN = 256
D = 512
D_HID = 1024
H = 8
DH = D // H
C = 96

CHUNK = 128
NCHUNK = N // CHUNK
CB = 2048
GB = 64
NEG = float("-inf")


def _mlp(x, W0, b0, Wa1, ba1, Wb1, bb1, Wa2, ba2, Wb2, bb2):
    h = jnp.dot(x, W0, preferred_element_type=jnp.float32) + b0
    z = jnp.maximum(jnp.dot(h, Wa1, preferred_element_type=jnp.float32) + ba1, 0.0)
    h = h + jnp.dot(z, Wb1, preferred_element_type=jnp.float32) + bb1
    z = jnp.maximum(jnp.dot(h, Wa2, preferred_element_type=jnp.float32) + ba2, 0.0)
    h = h + jnp.dot(z, Wb2, preferred_element_type=jnp.float32) + bb2
    return h


def _a_body(x, W0, b0, Wa1, ba1, Wb1, bb1, Wa2, ba2, Wb2, bb2,
            Wq, Wk, Wv, Wlbl, blbl,
            q_o, qh_o, wlk_o, wlv_o, bk_o, bv_o):
    q = _mlp(x[...], W0[...], b0[...], Wa1[...], ba1[...], Wb1[...],
             bb1[...], Wa2[...], ba2[...], Wb2[...], bb2[...])
    q_o[...] = q
    qh_o[...] = jnp.dot(q, Wq[...], preferred_element_type=jnp.float32)
    wlk_o[...] = jnp.dot(Wlbl[...], Wk[...], preferred_element_type=jnp.float32)
    wlv_o[...] = jnp.dot(Wlbl[...], Wv[...], preferred_element_type=jnp.float32)
    bk_o[...] = jnp.dot(blbl[...], Wk[...], preferred_element_type=jnp.float32)
    bv_o[...] = jnp.dot(blbl[...], Wv[...], preferred_element_type=jnp.float32)


def _run_a(x_num, W0, b0, Wa1, ba1, Wb1, bb1, Wa2, ba2, Wb2, bb2,
           Wq, Wk, Wv, Wlbl, blbl):
    outs = [
        jax.ShapeDtypeStruct((B, D), jnp.float32),
        jax.ShapeDtypeStruct((B, D), jnp.float32),
        jax.ShapeDtypeStruct((1, D), jnp.float32),
        jax.ShapeDtypeStruct((1, D), jnp.float32),
        jax.ShapeDtypeStruct((1, D), jnp.float32),
        jax.ShapeDtypeStruct((1, D), jnp.float32),
    ]
    return pl.pallas_call(_a_body, out_shape=outs)(
        x_num, W0, b0, Wa1, ba1, Wb1, bb1, Wa2, ba2, Wb2, bb2,
        Wq, Wk, Wv, Wlbl, blbl)


def _b_body(xc, q, W0, b0, Wa1, ba1, Wb1, bb1, Wa2, ba2, Wb2, bb2,
            kc_o, sims_o, cmax_o):
    kc = _mlp(xc[...], W0[...], b0[...], Wa1[...], ba1[...], Wb1[...],
              bb1[...], Wa2[...], ba2[...], Wb2[...], bb2[...])
    kc_o[...] = kc
    sims = jax.lax.dot_general(q[...], kc, (((1,), (1,)), ((), ())),
                               preferred_element_type=jnp.float32)
    sims_o[...] = sims
    cmax_o[...] = jnp.max(sims.reshape(B, CB // CHUNK, CHUNK), axis=-1)[None]


def _run_b(candidate_x, q, W0, b0, Wa1, ba1, Wb1, bb1, Wa2, ba2, Wb2, bb2):
    nsteps = N // CB
    full = lambda r, c: pl.BlockSpec((r, c), lambda i: (0, 0))
    in_specs = [
        pl.BlockSpec((CB, D_IN), lambda i: (i, 0)),
        full(B, D),
        full(D_IN, D), full(1, D),
        full(D, D_HID), full(1, D_HID), full(D_HID, D), full(1, D),
        full(D, D_HID), full(1, D_HID), full(D_HID, D), full(1, D),
    ]
    out_specs = [
        pl.BlockSpec((CB, D), lambda i: (i, 0)),
        pl.BlockSpec((B, CB), lambda i: (0, i)),
        pl.BlockSpec((1, B, CB // CHUNK), lambda i: (i, 0, 0)),
    ]
    outs = [
        jax.ShapeDtypeStruct((N, D), jnp.float32),
        jax.ShapeDtypeStruct((B, N), jnp.float32),
        jax.ShapeDtypeStruct((nsteps, B, CB // CHUNK), jnp.float32),
    ]
    return pl.pallas_call(
        _b_body, grid=(nsteps,), in_specs=in_specs, out_specs=out_specs,
        out_shape=outs,
        compiler_params=pltpu.CompilerParams(
            dimension_semantics=("arbitrary",)),
    )(candidate_x, q, W0, b0, Wa1, ba1, Wb1, bb1, Wa2, ba2, Wb2, bb2)


def _topc_body(W, vals_ref, gid_o):
    cols = jax.lax.broadcasted_iota(jnp.int32, (B, W), 1)
    rows = jax.lax.broadcasted_iota(jnp.int32, (B, 1), 0)
    colsC = jax.lax.broadcasted_iota(jnp.int32, (B, C), 1)

    def step(t, carry):
        vals, acc = carry
        m = jnp.max(vals, axis=1, keepdims=True)
        eq = vals == m
        pos = jnp.min(jnp.where(eq, cols, W), axis=1, keepdims=True)
        acc = jnp.where(colsC == t, pos + rows * W, acc)
        return (jnp.where(cols == pos, NEG, vals), acc)

    acc0 = jnp.zeros((B, C), jnp.int32)
    _, acc = jax.lax.fori_loop(0, C, step, (vals_ref[...], acc0))
    gid_o[...] = acc


def _run_topc(vals, W):
    return pl.pallas_call(
        functools.partial(_topc_body, W),
        out_shape=jax.ShapeDtypeStruct((B, C), jnp.int32))(vals)


S = 256
NW = 32
RW = B // NW


def _dl_body(sims_v, gids_f, gv_o, gid_v, chunk_v, sem):
    wid = lax.axis_index("s") * 2 + lax.axis_index("c")
    r0 = wid * RW

    def row_step(i, _):
        b = r0 + i
        pltpu.sync_copy(gids_f.at[pl.ds(b * C, C)], gid_v)
        pltpu.async_copy(sims_v.at[gid_v], chunk_v, sem).wait()
        pltpu.sync_copy(chunk_v, gv_o.at[pl.ds(b * C, C)])
        return 0

    lax.fori_loop(0, RW, row_step, 0)


def _run_dl(sims_v, gids_f):
    mesh = plsc.VectorSubcoreMesh(core_axis_name="c", subcore_axis_name="s")
    f = pl.kernel(
        _dl_body, mesh=mesh,
        out_type=jax.ShapeDtypeStruct((B * C, CHUNK), jnp.float32),
        scratch_types=[
            pltpu.VMEM((C,), jnp.int32),
            pltpu.VMEM((C, CHUNK), jnp.float32),
            pltpu.SemaphoreType.DMA,
        ])
    return f(sims_v, gids_f)


NSUB = C * CHUNK // 16
EQ = 64


def _e1_body(gv, gid3, smax_o, n3_o):
    j = pl.program_id(0)
    g = gv[...].reshape(EQ, NSUB, 16)
    smax_o[...] = jnp.max(g, axis=-1)
    rowg = jax.lax.broadcasted_iota(jnp.int32, (EQ, C), 0) + j * EQ
    cid2 = gid3[0] - rowg * NCHUNK
    n3_o[...] = (cid2[:, :, None] * CHUNK
                 + jax.lax.broadcasted_iota(jnp.int32, (EQ, C, CHUNK), 2))


def _run_e1(gv2, gid3):
    nsteps = B // EQ
    in_specs = [
        pl.BlockSpec((EQ, C * CHUNK), lambda j: (j, 0)),
        pl.BlockSpec((1, EQ, C), lambda j: (j, 0, 0)),
    ]
    out_specs = [
        pl.BlockSpec((EQ, NSUB), lambda j: (j, 0)),
        pl.BlockSpec((EQ, C, CHUNK), lambda j: (j, 0, 0)),
    ]
    outs = [
        jax.ShapeDtypeStruct((B, NSUB), jnp.float32),
        jax.ShapeDtypeStruct((B, C, CHUNK), jnp.int32),
    ]
    return pl.pallas_call(
        _e1_body, grid=(nsteps,), in_specs=in_specs, out_specs=out_specs,
        out_shape=outs,
        compiler_params=pltpu.CompilerParams(
            dimension_semantics=("arbitrary",)),
    )(gv2, gid3)


KJ2 = 128
FIRE = 12
KO2 = KJ2 * FIRE
BPW2 = (B * C * 16) // NW


def _d2_body(vt, nt, idx16, sv_o, sn_o, idx_v, v_v, n_v, sem, sem2):
    wid = lax.axis_index("s") * 2 + lax.axis_index("c")

    def step(t, _):
        base = wid * BPW2 + t * KO2
        pltpu.sync_copy(idx16.at[pl.ds(base, KO2)], idx_v)
        cps = []
        for bk in range(FIRE):
            sl = pl.ds(bk * KJ2, KJ2)
            cps.append(pltpu.async_copy(vt.at[idx_v.at[sl]], v_v.at[sl], sem))
            cps.append(pltpu.async_copy(nt.at[idx_v.at[sl]], n_v.at[sl], sem2))
        for cp in cps:
            cp.wait()
        pltpu.sync_copy(v_v, sv_o.at[pl.ds(base, KO2)])
        pltpu.sync_copy(n_v, sn_o.at[pl.ds(base, KO2)])
        return 0

    lax.fori_loop(0, BPW2 // KO2, step, 0)


def _run_d2(vt, nt, idx16):
    mesh = plsc.VectorSubcoreMesh(core_axis_name="c", subcore_axis_name="s")
    f = pl.kernel(
        _d2_body, mesh=mesh,
        out_type=[
            jax.ShapeDtypeStruct((B * C * 16,), jnp.float32),
            jax.ShapeDtypeStruct((B * C * 16,), jnp.int32),
        ],
        scratch_types=[
            pltpu.VMEM((KO2,), jnp.int32),
            pltpu.VMEM((KO2,), jnp.float32),
            pltpu.VMEM((KO2,), jnp.int32),
            pltpu.SemaphoreType.DMA,
            pltpu.SemaphoreType.DMA,
        ])
    return f(vt, nt, idx16)


SS = C * 16


def _e3_body(sv, sn, idx_o):
    cols = jax.lax.broadcasted_iota(jnp.int32, (B, SS), 1)
    colsC = jax.lax.broadcasted_iota(jnp.int32, (B, C), 1)
    snv = sn[...]

    def step(t, carry):
        vals, acc = carry
        m = jnp.max(vals, axis=1, keepdims=True)
        eq = vals == m
        pos = jnp.min(jnp.where(eq, cols, SS), axis=1, keepdims=True)
        sel = jnp.min(jnp.where(cols == pos, snv, N), axis=1, keepdims=True)
        acc = jnp.where(colsC == t, sel, acc)
        return (jnp.where(cols == pos, NEG, vals), acc)

    acc0 = jnp.zeros((B, C), jnp.int32)
    _, acc = jax.lax.fori_loop(0, C, step, (sv[...], acc0))
    idx_o[...] = acc


def _run_e3(sv, sn):
    return pl.pallas_call(
        _e3_body,
        out_shape=jax.ShapeDtypeStruct((B, C), jnp.int32))(sv, sn)


KJ = 96
BPW = (B * C) // NW


FIREF = 2


def _f_body(kc, y, idxf, ctx_o, yc_o, idx_v, rows_v, y_v, sem, sem2):
    wid = lax.axis_index("s") * 2 + lax.axis_index("c")

    def step(t, _):
        base = wid * BPW + t * KJ * FIREF
        pltpu.sync_copy(idxf.at[pl.ds(base, KJ * FIREF)], idx_v)
        cps = []
        for bk in range(FIREF):
            sl = pl.ds(bk * KJ, KJ)
            cps.append(pltpu.async_copy(kc.at[idx_v.at[sl]],
                                        rows_v.at[sl], sem))
            cps.append(pltpu.async_copy(y.at[idx_v.at[sl]],
                                        y_v.at[sl], sem2))
        for cp in cps:
            cp.wait()
        pltpu.sync_copy(rows_v, ctx_o.at[pl.ds(base, KJ * FIREF)])
        pltpu.sync_copy(y_v, yc_o.at[pl.ds(base, KJ * FIREF)])
        return 0

    lax.fori_loop(0, BPW // (KJ * FIREF), step, 0)


def _run_f(kc, y, idxf):
    mesh = plsc.VectorSubcoreMesh(core_axis_name="c", subcore_axis_name="s")
    f = pl.kernel(
        _f_body, mesh=mesh,
        out_type=[
            jax.ShapeDtypeStruct((B * C, D), jnp.float32),
            jax.ShapeDtypeStruct((B * C,), jnp.float32),
        ],
        scratch_types=[
            pltpu.VMEM((KJ * FIREF,), jnp.int32),
            pltpu.VMEM((KJ * FIREF, D), jnp.float32),
            pltpu.VMEM((KJ * FIREF,), jnp.float32),
            pltpu.SemaphoreType.DMA,
            pltpu.SemaphoreType.DMA,
        ])
    return f(kc, y, idxf)


def _g_body(ctx3, y2, qh, q, wlk, wlv, bk, bv, Wk, Wv, Wo, Wh, bh, out_o):
    ctx = ctx3[...].reshape(C * GB, D)
    kh0 = jnp.dot(ctx, Wk[...], preferred_element_type=jnp.float32) + bk[...]
    vh0 = jnp.dot(ctx, Wv[...], preferred_element_type=jnp.float32) + bv[...]

    r0 = jax.lax.broadcasted_iota(jnp.int32, (C * GB, GB), 0)
    r1 = jax.lax.broadcasted_iota(jnp.int32, (C * GB, GB), 1)
    R = (r0 % GB == r1).astype(jnp.float32)
    g0 = jax.lax.broadcasted_iota(jnp.int32, (D, H), 0)
    g1 = jax.lax.broadcasted_iota(jnp.int32, (D, H), 1)
    Gm = (g0 // DH == g1).astype(jnp.float32)
    gt0 = jax.lax.broadcasted_iota(jnp.int32, (H, D), 0)
    gt1 = jax.lax.broadcasted_iota(jnp.int32, (H, D), 1)
    GmT = (gt1 // DH == gt0).astype(jnp.float32)
    rt0 = jax.lax.broadcasted_iota(jnp.int32, (GB, C * GB), 0)
    rt1 = jax.lax.broadcasted_iota(jnp.int32, (GB, C * GB), 1)
    RT = (rt1 % GB == rt0).astype(jnp.float32)

    qhv = qh[...]
    qh_exp = jnp.dot(R, qhv, preferred_element_type=jnp.float32)
    L0 = jnp.dot(kh0 * qh_exp, Gm, preferred_element_type=jnp.float32)
    A_ = jnp.dot(qhv * wlk[...], Gm, preferred_element_type=jnp.float32)
    A_exp = jnp.dot(R, A_, preferred_element_type=jnp.float32)
    y3 = y2[0]
    Ly = (y3[:, :, None] * A_exp.reshape(C, GB, H)).reshape(C * GB, H)
    L = (L0 + Ly) * (1.0 / np.sqrt(DH))

    L3 = L.reshape(C, GB, H)
    M = jnp.max(L3, axis=0)
    P = jnp.exp(L3 - M[None])
    Ssum = jnp.sum(P, axis=0)
    W3 = P / Ssum[None]
    W2 = W3.reshape(C * GB, H)
    w_exp = jnp.dot(W2, GmT, preferred_element_type=jnp.float32)
    wy = (w_exp.reshape(C, GB, D) * y3[:, :, None]).reshape(C * GB, D)
    Z = w_exp * vh0 + wy * wlv[...]
    o = jnp.dot(RT, Z, preferred_element_type=jnp.float32)
    hq = q[...] + jnp.dot(o, Wo[...], preferred_element_type=jnp.float32)
    out_o[...] = jnp.dot(hq, Wh[...], preferred_element_type=jnp.float32) + bh[...]


def _run_g(ctx3, y2, qh, q, wlk, wlv, bk, bv, Wk, Wv, Wo, Wh, bh):
    nsteps = B // GB
    full = lambda r, c: pl.BlockSpec((r, c), lambda j: (0, 0))
    in_specs = [
        pl.BlockSpec((C, GB, D), lambda j: (0, j, 0)),
        pl.BlockSpec((1, C, GB), lambda j: (j, 0, 0)),
        pl.BlockSpec((GB, D), lambda j: (j, 0)),
        pl.BlockSpec((GB, D), lambda j: (j, 0)),
        full(1, D), full(1, D), full(1, D), full(1, D),
        full(D, D), full(D, D), full(D, D), full(D, 1), full(1, 1),
    ]
    return pl.pallas_call(
        _g_body, grid=(nsteps,), in_specs=in_specs,
        out_specs=pl.BlockSpec((GB, 1), lambda j: (j, 0)),
        out_shape=jax.ShapeDtypeStruct((B, 1), jnp.float32),
        compiler_params=pltpu.CompilerParams(
            dimension_semantics=("arbitrary",)),
    )(ctx3, y2, qh, q, wlk, wlv, bk, bv, Wk, Wv, Wo, Wh, bh)


def kernel(x_num, candidate_x, candidate_y, W0, b0, Wa1, ba1, Wb1, bb1,
           Wa2, ba2, Wb2, bb2, Wlbl, blbl, Wq, Wk, Wv, Wo, Wh, bh):
    r2 = lambda v: v.reshape(1, -1)
    b0r, ba1r, bb1r, ba2r, bb2r = r2(b0), r2(ba1), r2(bb1), r2(ba2), r2(bb2)

    q, qh, wlk, wlv, bk, bv = _run_a(
        x_num, W0, b0r, Wa1, ba1r, Wb1, bb1r, Wa2, ba2r, Wb2, bb2r,
        Wq, Wk, Wv, Wlbl, r2(blbl))

    kc, sims, cmax3 = _run_b(
        candidate_x, q, W0, b0r, Wa1, ba1r, Wb1, bb1r, Wa2, ba2r, Wb2, bb2r)
    cmax = cmax3.transpose(1, 0, 2).reshape(B, NCHUNK)

    gids = _run_topc(cmax, NCHUNK)
    gv = _run_dl(sims.reshape(B * NCHUNK, CHUNK), gids.reshape(-1))
    smax, n3 = _run_e1(gv.reshape(B, C * CHUNK),
                       gids.reshape(B // EQ, EQ, C))
    sgids = _run_topc(smax, NSUB)
    idx16 = (sgids[:, :, None] * 16
             + jnp.arange(16, dtype=jnp.int32)).reshape(-1)
    sv, sn = _run_d2(gv.reshape(-1), n3.reshape(-1), idx16)
    idx = _run_e3(sv.reshape(B, SS), sn.reshape(B, SS))
    idxf = idx.T.reshape(-1)
    ctx_flat, yc = _run_f(kc, candidate_y, idxf)
    ctx3 = ctx_flat.reshape(C, B, D)
    y2 = yc.reshape(C, B // GB, GB).transpose(1, 0, 2)

    return _run_g(ctx3, y2, qh, q, wlk, wlv, bk, bv, Wk, Wv, Wo, Wh,
                  r2(bh))
